# baseline (speedup 1.0000x reference)
import functools

import jax
import jax.numpy as jnp
from jax import lax
from jax.experimental import pallas as pl
from jax.experimental.pallas import tpu as pltpu
from jax.experimental.pallas import tpu_sc as plsc

E = 8
TOPK = 2
D = 2048
DFF = 4096
T = 8192
TE = T // E


def _rne_bf16_hi(b):
    return b + jnp.int32(0x7FFF) + (lax.shift_right_logical(b, 16)
                                    & jnp.int32(1))


BF = 512
NF1 = DFF // BF
BD = 512
ND = D // BD
NP = NF1 + ND


def _mlp_body(x_ref, gate_ref, w1_ref, w2_ref, out_ref, h_ref):
    p = pl.program_id(1)

    @pl.when(p < NF1)
    def _():
        h = lax.dot_general(x_ref[...], w1_ref[0], (((1,), (1,)), ((), ())),
                            preferred_element_type=jnp.float32)
        h_ref[p] = jax.nn.gelu(h).astype(jnp.bfloat16)

    @pl.when(p >= NF1)
    def _():
        y = jnp.zeros((TE, BD), jnp.float32)
        for f in range(NF1):
            y = y + lax.dot_general(
                h_ref[f], w2_ref[0][:, f * BF:(f + 1) * BF],
                (((1,), (1,)), ((), ())),
                preferred_element_type=jnp.float32)
        yg = y * gate_ref[...]
        b0 = lax.bitcast_convert_type(yg[:, :BD // 2], jnp.int32)
        b1 = lax.bitcast_convert_type(yg[:, BD // 2:], jnp.int32)
        lo = lax.shift_right_logical(_rne_bf16_hi(b0), 16)
        hi = _rne_bf16_hi(b1) & jnp.int32(-65536)
        out_ref[...] = hi | lo


def _expert_mlp(x, gate2d, W1, W2):
    return pl.pallas_call(
        _mlp_body,
        grid=(E, NP),
        in_specs=[
            pl.BlockSpec((TE, D), lambda e, p: (e, 0)),
            pl.BlockSpec((TE, 1), lambda e, p: (e, 0)),
            pl.BlockSpec((1, BF, D),
                         lambda e, p: (e, jnp.minimum(p, NF1 - 1), 0)),
            pl.BlockSpec((1, BD, DFF),
                         lambda e, p: (e, jnp.maximum(p - NF1, 0), 0)),
        ],
        out_specs=pl.BlockSpec((TE, BD // 2),
                               lambda e, p: (e, jnp.maximum(p - NF1, 0))),
        out_shape=jax.ShapeDtypeStruct((T, D // 2), jnp.int32),
        scratch_shapes=[pltpu.VMEM((NF1, TE, BF), jnp.bfloat16)],
        compiler_params=pltpu.CompilerParams(
            dimension_semantics=("parallel", "arbitrary")),
    )(x, gate2d, W1, W2)


NW = 32
IPW = T // NW
CH = 16
NCHK = IPW // CH


NBUF = 6


def _scatter_body(res_hbm, nidx_hbm, full_hbm, nidx_v, idx_v, *bufsem):
    bufs = bufsem[:NBUF]
    lsems = bufsem[NBUF:2 * NBUF]
    ssems = bufsem[2 * NBUF:]
    w = lax.axis_index("c") * 16 + lax.axis_index("s")
    base = w * IPW
    pltpu.sync_copy(nidx_hbm.at[pl.ds(base, IPW)], nidx_v)
    for i in range(NCHK):
        v = nidx_v[pl.ds(i * CH, CH)]
        idx_v[i, ...] = ((v & 1) << 12) | lax.shift_right_logical(v, 1)
    loads = [None] * NCHK
    scats = [None] * NCHK
    for ch in range(min(NBUF, NCHK)):
        loads[ch] = pltpu.async_copy(
            res_hbm.at[pl.ds(base + ch * CH, CH)], bufs[ch], lsems[ch])
    for ch in range(NCHK):
        b = ch % NBUF
        loads[ch].wait()
        scats[ch] = pltpu.async_copy(
            bufs[b], full_hbm.at[idx_v.at[ch]], ssems[b])
        nxt = ch + 1
        if NBUF <= nxt < NCHK:
            scats[nxt - NBUF].wait()
            loads[nxt] = pltpu.async_copy(
                res_hbm.at[pl.ds(base + nxt * CH, CH)], bufs[nxt % NBUF],
                lsems[nxt % NBUF])
    for ch in range(max(0, NCHK - NBUF), NCHK):
        scats[ch].wait()


@functools.partial(
    pl.kernel,
    out_type=jax.ShapeDtypeStruct((T, D // 2), jnp.int32),
    mesh=plsc.VectorSubcoreMesh(core_axis_name="c", subcore_axis_name="s"),
    scratch_types=(
        [pltpu.VMEM((IPW,), jnp.int32),
         pltpu.VMEM((NCHK, CH), jnp.int32)]
        + [pltpu.VMEM((CH, D // 2), jnp.int32) for _ in range(NBUF)]
        + [pltpu.SemaphoreType.DMA for _ in range(2 * NBUF)]
    ),
)
def _scatter(res_hbm, nidx_hbm, full_hbm, nidx_v, idx_v, *bufsem):
    _scatter_body(res_hbm, nidx_hbm, full_hbm, nidx_v, idx_v, *bufsem)


BT = 512
NBT = (T // TOPK) // BT


QW = BD // 2


def _unpack_lo(a):
    return lax.bitcast_convert_type(lax.shift_left(a, 16), jnp.float32)


def _unpack_hi(a):
    return lax.bitcast_convert_type(a & jnp.int32(-65536), jnp.float32)


def _pairsum_body(a_ref, b_ref, out_ref):
    for q in range(ND):
        a = a_ref[:, pl.ds(q * QW, QW)]
        b = b_ref[:, pl.ds(q * QW, QW)]
        out_ref[:, pl.ds(q * BD, QW)] = _unpack_lo(a) + _unpack_lo(b)
        out_ref[:, pl.ds(q * BD + QW, QW)] = _unpack_hi(a) + _unpack_hi(b)


def _pairsum(full):
    return pl.pallas_call(
        _pairsum_body,
        grid=(NBT,),
        in_specs=[
            pl.BlockSpec((BT, D // 2), lambda i: (i, 0)),
            pl.BlockSpec((BT, D // 2), lambda i: (i + NBT, 0)),
        ],
        out_specs=pl.BlockSpec((BT, D), lambda i: (i, 0)),
        out_shape=jax.ShapeDtypeStruct((T // TOPK, D), jnp.float32),
        compiler_params=pltpu.CompilerParams(
            dimension_semantics=("arbitrary",)),
    )(full, full)


def kernel(inputs_shard, gate_weight, choosed_experts, new_index, W1, W2):
    gate2d = gate_weight.reshape(T, 1)
    result = _expert_mlp(inputs_shard, gate2d, W1, W2)
    full = _scatter(result, new_index)
    out2 = _pairsum(full)
    mlp_bias = jnp.zeros((D,), dtype=out2.dtype)
    return (out2, mlp_bias)


# speedup vs baseline: 1.0393x; 1.0393x over previous
import functools

import jax
import jax.numpy as jnp
from jax import lax
from jax.experimental import pallas as pl
from jax.experimental.pallas import tpu as pltpu
from jax.experimental.pallas import tpu_sc as plsc

E = 8
TOPK = 2
D = 2048
DFF = 4096
T = 8192
TE = T // E


def _fc1_body(x_ref, w1_ref, h_ref):
    h = lax.dot_general(x_ref[...], w1_ref[0], (((1,), (1,)), ((), ())),
                        preferred_element_type=jnp.float32)
    h_ref[...] = jax.nn.gelu(h).astype(jnp.bfloat16)


def _rne_bf16_hi(b):
    return b + jnp.int32(0x7FFF) + (lax.shift_right_logical(b, 16)
                                    & jnp.int32(1))


def _fc2_body(h_ref, gate_ref, w2_ref, out_ref):
    y = lax.dot_general(h_ref[...], w2_ref[0], (((1,), (1,)), ((), ())),
                        preferred_element_type=jnp.float32)
    yg = y * gate_ref[...]
    b0 = lax.bitcast_convert_type(yg[:, :BD // 2], jnp.int32)
    b1 = lax.bitcast_convert_type(yg[:, BD // 2:], jnp.int32)
    lo = lax.shift_right_logical(_rne_bf16_hi(b0), 16)
    hi = _rne_bf16_hi(b1) & jnp.int32(-65536)
    out_ref[...] = hi | lo


BF = 2048
NF1 = DFF // BF
BD = 512
ND = D // BD


def _expert_mlp(x, gate2d, W1, W2):
    h = pl.pallas_call(
        _fc1_body,
        grid=(E, NF1),
        in_specs=[
            pl.BlockSpec((TE, D), lambda e, f: (e, 0)),
            pl.BlockSpec((1, BF, D), lambda e, f: (e, f, 0)),
        ],
        out_specs=pl.BlockSpec((TE, BF), lambda e, f: (e, f)),
        out_shape=jax.ShapeDtypeStruct((T, DFF), jnp.bfloat16),
        compiler_params=pltpu.CompilerParams(
            dimension_semantics=("parallel", "arbitrary")),
    )(x, W1)
    return pl.pallas_call(
        _fc2_body,
        grid=(E, ND),
        in_specs=[
            pl.BlockSpec((TE, DFF), lambda e, d: (e, 0)),
            pl.BlockSpec((TE, 1), lambda e, d: (e, 0)),
            pl.BlockSpec((1, BD, DFF), lambda e, d: (e, d, 0)),
        ],
        out_specs=pl.BlockSpec((TE, BD // 2), lambda e, d: (e, d)),
        out_shape=jax.ShapeDtypeStruct((T, D // 2), jnp.int32),
        compiler_params=pltpu.CompilerParams(
            dimension_semantics=("parallel", "arbitrary")),
    )(h, gate2d, W2)


NW = 32
IPW = T // NW
CH = 16
NCHK = IPW // CH


NBUF = 6


def _scatter_body(res_hbm, nidx_hbm, full_hbm, nidx_v, idx_v, *bufsem):
    bufs = bufsem[:NBUF]
    lsems = bufsem[NBUF:2 * NBUF]
    ssems = bufsem[2 * NBUF:]
    w = lax.axis_index("c") * 16 + lax.axis_index("s")
    base = w * IPW
    pltpu.sync_copy(nidx_hbm.at[pl.ds(base, IPW)], nidx_v)
    for i in range(NCHK):
        v = nidx_v[pl.ds(i * CH, CH)]
        idx_v[i, ...] = ((v & 1) << 12) | lax.shift_right_logical(v, 1)
    loads = [None] * NCHK
    scats = [None] * NCHK
    for ch in range(min(NBUF, NCHK)):
        loads[ch] = pltpu.async_copy(
            res_hbm.at[pl.ds(base + ch * CH, CH)], bufs[ch], lsems[ch])
    for ch in range(NCHK):
        b = ch % NBUF
        loads[ch].wait()
        scats[ch] = pltpu.async_copy(
            bufs[b], full_hbm.at[idx_v.at[ch]], ssems[b])
        nxt = ch + 1
        if NBUF <= nxt < NCHK:
            scats[nxt - NBUF].wait()
            loads[nxt] = pltpu.async_copy(
                res_hbm.at[pl.ds(base + nxt * CH, CH)], bufs[nxt % NBUF],
                lsems[nxt % NBUF])
    for ch in range(max(0, NCHK - NBUF), NCHK):
        scats[ch].wait()


@functools.partial(
    pl.kernel,
    out_type=jax.ShapeDtypeStruct((T, D // 2), jnp.int32),
    mesh=plsc.VectorSubcoreMesh(core_axis_name="c", subcore_axis_name="s"),
    scratch_types=(
        [pltpu.VMEM((IPW,), jnp.int32),
         pltpu.VMEM((NCHK, CH), jnp.int32)]
        + [pltpu.VMEM((CH, D // 2), jnp.int32) for _ in range(NBUF)]
        + [pltpu.SemaphoreType.DMA for _ in range(2 * NBUF)]
    ),
)
def _scatter(res_hbm, nidx_hbm, full_hbm, nidx_v, idx_v, *bufsem):
    _scatter_body(res_hbm, nidx_hbm, full_hbm, nidx_v, idx_v, *bufsem)


BT = 512
NBT = (T // TOPK) // BT


QW = BD // 2


def _unpack_lo(a):
    return lax.bitcast_convert_type(lax.shift_left(a, 16), jnp.float32)


def _unpack_hi(a):
    return lax.bitcast_convert_type(a & jnp.int32(-65536), jnp.float32)


def _pairsum_body(a_ref, b_ref, out_ref):
    for q in range(ND):
        a = a_ref[:, pl.ds(q * QW, QW)]
        b = b_ref[:, pl.ds(q * QW, QW)]
        out_ref[:, pl.ds(q * BD, QW)] = _unpack_lo(a) + _unpack_lo(b)
        out_ref[:, pl.ds(q * BD + QW, QW)] = _unpack_hi(a) + _unpack_hi(b)


def _pairsum(full):
    return pl.pallas_call(
        _pairsum_body,
        grid=(NBT,),
        in_specs=[
            pl.BlockSpec((BT, D // 2), lambda i: (i, 0)),
            pl.BlockSpec((BT, D // 2), lambda i: (i + NBT, 0)),
        ],
        out_specs=pl.BlockSpec((BT, D), lambda i: (i, 0)),
        out_shape=jax.ShapeDtypeStruct((T // TOPK, D), jnp.float32),
        compiler_params=pltpu.CompilerParams(
            dimension_semantics=("arbitrary",)),
    )(full, full)


def kernel(inputs_shard, gate_weight, choosed_experts, new_index, W1, W2):
    gate2d = gate_weight.reshape(T, 1)
    result = _expert_mlp(inputs_shard, gate2d, W1, W2)
    full = _scatter(result, new_index)
    out2 = _pairsum(full)
    mlp_bias = jnp.zeros((D,), dtype=out2.dtype)
    return (out2, mlp_bias)
